# Initial kernel scaffold; baseline (speedup 1.0000x reference)
import jax
import jax.numpy as jnp
from jax.experimental import pallas as pl


def kernel(x, edge_index, edge_feats, W0, b0, W1, b1, W2, b2, W3, b3, W4, b4, ln1_g, ln1_b, ln2_g, ln2_b, lin_W, lin_b):
    raise NotImplementedError("write your pallas kernel here")


# speedup vs baseline: 7.5337x; 7.5337x over previous
import functools

import jax
import jax.numpy as jnp
from jax import lax
from jax.experimental import pallas as pl
from jax.experimental.pallas import tpu as pltpu
from jax.experimental.pallas import tpu_sc as plsc

N = 10000
E = 320000
D = 128

NC = 2
NS = 16
NW = NC * NS
EPW = E // NW
CHUNK = 80
NCHUNK = EPW // CHUNK
RPT = 624
REM = N - NS * RPT
DEGR = 80

@functools.lru_cache(maxsize=None)
def _sc_mesh():
  return plsc.VectorSubcoreMesh(
      core_axis_name="c", subcore_axis_name="s", num_cores=NC, num_subcores=NS)


def _zero_vmem_rows(ref, nrows):
  def body(r, _):
    for j in range(D // 16):
      ref[r, pl.ds(16 * j, 16)] = jnp.zeros((16,), jnp.float32)
    return 0
  lax.fori_loop(0, nrows, body, 0)


def _deg_body(dst_hbm, ew_hbm, out_hbm, dst_v, ew_v, degflat, deg2d, idx_v,
              zbuf, acc_sh):
  cid = lax.axis_index("c")
  sid = lax.axis_index("s")
  wid = sid * NC + cid
  pltpu.sync_copy(dst_hbm.at[pl.ds(wid * EPW, EPW)], dst_v)
  pltpu.sync_copy(ew_hbm.at[pl.ds(wid * EPW, EPW)], ew_v)

  def zb(i, _):
    degflat[pl.ds(i * 16, 16)] = jnp.zeros((16,), jnp.float32)
    return 0
  lax.fori_loop(0, DEGR * D // 16, zb, 0)
  _zero_vmem_rows(zbuf, 8)
  @pl.when(sid < 10)
  def _():
    pltpu.sync_copy(zbuf, acc_sh.at[pl.ds(sid * 8, 8)])
  for k in range(DEGR // 16):
    idx_v[pl.ds(16 * k, 16)] = lax.iota(jnp.int32, 16) + 16 * k

  def body(i, _):
    d16 = dst_v[pl.ds(i * 16, 16)]
    w16 = ew_v[pl.ds(i * 16, 16)]
    plsc.addupdate_scatter(degflat, [d16], w16)
    return 0
  lax.fori_loop(0, EPW // 16, body, 0)

  def repack(r, _):
    for j in range(D // 16):
      deg2d[r, pl.ds(16 * j, 16)] = degflat[pl.ds(r * D + 16 * j, 16)]
    return 0
  lax.fori_loop(0, DEGR, repack, 0)

  plsc.subcore_barrier()
  pltpu.sync_copy(deg2d, acc_sh.at[idx_v], add=True)
  plsc.subcore_barrier()

  @pl.when(sid == 0)
  def _():
    pltpu.sync_copy(acc_sh, out_hbm.at[cid])


@functools.lru_cache(maxsize=None)
def _deg_call():
  return pl.kernel(
    _deg_body,
    out_type=jax.ShapeDtypeStruct((NC, DEGR, D), jnp.float32),
    mesh=_sc_mesh(),
    compiler_params=pltpu.CompilerParams(needs_layout_passes=False),
    scratch_types=[
        pltpu.VMEM((EPW,), jnp.int32),
        pltpu.VMEM((EPW,), jnp.float32),
        pltpu.VMEM((DEGR * D,), jnp.float32),
        pltpu.VMEM((DEGR, D), jnp.float32),
        pltpu.VMEM((DEGR,), jnp.int32),
        pltpu.VMEM((8, D), jnp.float32),
        pltpu.VMEM_SHARED((DEGR, D), jnp.float32),
    ],
  )


def _agg_body(mp_hbm, src_hbm, dst_hbm, ew_hbm, out_hbm,
              src_c, dst_c, ew_c, buf, acc_sh):
  cid = lax.axis_index("c")
  sid = lax.axis_index("s")
  wid = sid * NC + cid
  base = wid * EPW

  _zero_vmem_rows(buf, CHUNK)
  for k in range(7):
    pltpu.sync_copy(buf, acc_sh.at[pl.ds(sid * RPT + k * CHUNK, CHUNK)])
  pltpu.sync_copy(buf.at[pl.ds(0, 64)],
                  acc_sh.at[pl.ds(sid * RPT + 7 * CHUNK, 64)])

  @pl.when(sid == NS - 1)
  def _():
    pltpu.sync_copy(buf.at[pl.ds(0, REM)], acc_sh.at[pl.ds(NS * RPT, REM)])
  plsc.subcore_barrier()

  def chunk(g, _):
    off = base + g * CHUNK
    pltpu.sync_copy(src_hbm.at[pl.ds(off, CHUNK)], src_c)
    pltpu.sync_copy(dst_hbm.at[pl.ds(off, CHUNK)], dst_c)
    pltpu.sync_copy(ew_hbm.at[pl.ds(off, CHUNK)], ew_c)
    pltpu.sync_copy(mp_hbm.at[src_c], buf)

    def row16(r16, carry):
      ew16 = ew_c[pl.ds(r16 * 16, 16)]
      for l in range(16):
        s = ew16[l]
        r = r16 * 16 + l
        for j in range(D // 16):
          buf[r, pl.ds(16 * j, 16)] = buf[r, pl.ds(16 * j, 16)] * s
      return carry
    lax.fori_loop(0, CHUNK // 16, row16, 0)
    pltpu.sync_copy(buf, acc_sh.at[dst_c], add=True)
    return 0
  lax.fori_loop(0, NCHUNK, chunk, 0)

  plsc.subcore_barrier()
  pltpu.sync_copy(acc_sh.at[pl.ds(sid * RPT, RPT)],
                  out_hbm.at[cid].at[pl.ds(sid * RPT, RPT)])

  @pl.when(sid == NS - 1)
  def _():
    pltpu.sync_copy(acc_sh.at[pl.ds(NS * RPT, REM)],
                    out_hbm.at[cid].at[pl.ds(NS * RPT, REM)])


@functools.lru_cache(maxsize=None)
def _agg_call():
  return pl.kernel(
    _agg_body,
    out_type=jax.ShapeDtypeStruct((NC, N, D), jnp.float32),
    mesh=_sc_mesh(),
    compiler_params=pltpu.CompilerParams(needs_layout_passes=False),
    scratch_types=[
        pltpu.VMEM((CHUNK,), jnp.int32),
        pltpu.VMEM((CHUNK,), jnp.int32),
        pltpu.VMEM((CHUNK,), jnp.float32),
        pltpu.VMEM((CHUNK, D), jnp.float32),
        pltpu.VMEM_SHARED((N, D), jnp.float32),
    ],
  )


BR = 1000


def _mm_scale_body(x_ref, w_ref, dinv_ref, o_ref):
  m = jnp.dot(x_ref[...], w_ref[...], preferred_element_type=jnp.float32)
  o_ref[...] = m * dinv_ref[...]


def _ln_elu(t, g, b):
  mu = jnp.mean(t, axis=-1, keepdims=True)
  var = jnp.mean((t - mu) ** 2, axis=-1, keepdims=True)
  ln = (t - mu) * lax.rsqrt(var + 1e-5) * g + b
  return jnp.where(ln > 0, ln, jnp.exp(ln) - 1.0)


def _layer_body(s0_ref, s1_ref, mp_ref, dinv_ref, b_ref, g_ref, lb_ref,
                w_ref, o_ref):
  t = (s0_ref[...] + s1_ref[...] + mp_ref[...]) * dinv_ref[...] + b_ref[...]
  h = _ln_elu(t, g_ref[...], lb_ref[...])
  o_ref[...] = jnp.dot(h, w_ref[...],
                       preferred_element_type=jnp.float32) * dinv_ref[...]


def _final_body(s0_ref, s1_ref, mp_ref, dinv_ref, b_ref, g_ref, lb_ref,
                w_ref, lb2_ref, o_ref):
  t = (s0_ref[...] + s1_ref[...] + mp_ref[...]) * dinv_ref[...] + b_ref[...]
  h = _ln_elu(t, g_ref[...], lb_ref[...])
  y = jnp.dot(h, w_ref[...], preferred_element_type=jnp.float32) + lb2_ref[...]
  o_ref[...] = jnp.maximum(y, 0.0)


_row_spec = pl.BlockSpec((BR, D), lambda i: (i, 0))
_dinv_spec = pl.BlockSpec((BR, 1), lambda i: (i, 0))
_vec_spec = pl.BlockSpec((1, D), lambda i: (0, 0))
_w_spec = pl.BlockSpec((D, D), lambda i: (0, 0))
_out_sds = jax.ShapeDtypeStruct((N, D), jnp.float32)
_grid = (N // BR,)

_mm_scale = pl.pallas_call(
    _mm_scale_body, grid=_grid,
    in_specs=[_row_spec, _w_spec, _dinv_spec],
    out_specs=_row_spec, out_shape=_out_sds)

_layer_tc = pl.pallas_call(
    _layer_body, grid=_grid,
    in_specs=[_row_spec, _row_spec, _row_spec, _dinv_spec,
              _vec_spec, _vec_spec, _vec_spec, _w_spec],
    out_specs=_row_spec, out_shape=_out_sds)

_final_tc = pl.pallas_call(
    _final_body, grid=_grid,
    in_specs=[_row_spec, _row_spec, _row_spec, _dinv_spec,
              _vec_spec, _vec_spec, _vec_spec, _w_spec, _vec_spec],
    out_specs=_row_spec, out_shape=_out_sds)


def kernel(x, edge_index, edge_feats, W0, b0, W1, b1, W2, b2, W3, b3, W4, b4,
           ln1_g, ln1_b, ln2_g, ln2_b, lin_W, lin_b):
  src = edge_index[0].astype(jnp.int32)
  dst = edge_index[1].astype(jnp.int32)
  ew = edge_feats.astype(jnp.float32)

  deg_p = _deg_call()(dst, ew)
  deg = (deg_p[0] + deg_p[1]).reshape(DEGR * D)[:N] + 1.0
  dinv = jnp.where(deg > 0, lax.rsqrt(deg), 0.0).reshape(N, 1)

  b1d = lambda v: v.reshape(1, D)
  g1, l1, g2, l2 = b1d(ln1_g), b1d(ln1_b), b1d(ln2_g), b1d(ln2_b)

  mp = _mm_scale(x, W0, dinv)
  Ws = [W1, W2, W3, W4]
  bs = [b0, b1, b2, b3]
  for i in range(4):
    S = _agg_call()(mp, src, dst, ew)
    mp = _layer_tc(S[0], S[1], mp, dinv, b1d(bs[i]), g1, l1, Ws[i])
  S = _agg_call()(mp, src, dst, ew)
  return _final_tc(S[0], S[1], mp, dinv, b1d(b4), g2, l2, lin_W, b1d(lin_b))
